# baseline (speedup 1.0000x reference)
import functools

import jax
import jax.numpy as jnp
from jax import lax
from jax.experimental import pallas as pl
from jax.experimental.pallas import tpu as pltpu
from jax.experimental.pallas import tpu_sc as plsc

N = 10000
E = 320000
IN_C = 128
HID = 128
OUT_C = 64

NC = 2
NS = 16
NW = NC * NS
K = 64
CHUNKS = 160
NBUF = 3
IQ = 4
EPW = K * CHUNKS
E_PAD = EPW * NW
NPAD = 10112
STRIPE = NPAD // NS
RB = 25
R = N // RB


def _p1_body(x_ref, w1_ref, att2_ref, h_ref, a_ref):
    h = jax.lax.dot_general(x_ref[...], w1_ref[...], (((1,), (0,)), ((), ())),
                            preferred_element_type=jnp.float32)
    h_ref[...] = h
    a_ref[...] = jax.lax.dot_general(h, att2_ref[...], (((1,), (0,)), ((), ())),
                                     preferred_element_type=jnp.float32)


def _phase1(x, W1, att2):
    return pl.pallas_call(
        _p1_body,
        grid=(RB,),
        in_specs=[
            pl.BlockSpec((R, IN_C), lambda i: (i, 0)),
            pl.BlockSpec((IN_C, HID), lambda i: (0, 0)),
            pl.BlockSpec((HID, 2), lambda i: (0, 0)),
        ],
        out_specs=[
            pl.BlockSpec((R, HID), lambda i: (i, 0)),
            pl.BlockSpec((R, 2), lambda i: (i, 0)),
        ],
        out_shape=[
            jax.ShapeDtypeStruct((N, HID), jnp.float32),
            jax.ShapeDtypeStruct((N, 2), jnp.float32),
        ],
    )(x, W1, att2)


def _sc_body(h_hbm, a_hbm, ed_hbm,
             outp_hbm, denp_hbm,
             tab_v, idx_v, w_v, rows_v, denbuf_v,
             out_sh, den_sh,
             *sems9):
    c = lax.axis_index("c")
    s = lax.axis_index("s")
    wid = c * NS + s
    sems = sems9[0:NBUF]
    rsems = sems9[NBUF:2 * NBUF]
    wsems = sems9[2 * NBUF:3 * NBUF]
    isems = sems9[3 * NBUF:3 * NBUF + IQ]

    pltpu.sync_copy(a_hbm, tab_v.at[pl.ds(0, 2 * N)])
    z16 = jnp.zeros((16,), jnp.float32)
    for t in range(2 * N, 2 * NPAD, 16):
        tab_v[pl.ds(t, 16)] = z16

    def _zrow(j, _):
        for g in range(HID // 16):
            rows_v[0, j, pl.ds(g * 16, 16)] = z16
        return 0
    lax.fori_loop(0, K, _zrow, 0)
    base = s * STRIPE
    for off in range(0, STRIPE, K):
        sz = min(K, STRIPE - off)
        pltpu.sync_copy(rows_v.at[0, pl.ds(0, sz)],
                        out_sh.at[pl.ds(base + off, sz)])
        pltpu.sync_copy(rows_v.at[0, 0, pl.ds(0, sz)],
                        den_sh.at[pl.ds(base + off, sz)])
    plsc.subcore_barrier()

    def _start_load_idx(ci, q):
        pltpu.async_copy(ed_hbm.at[wid * CHUNKS + ci], idx_v.at[q], isems[q])

    def _wait_load_idx(ci, q):
        pltpu.make_async_copy(ed_hbm.at[wid * CHUNKS + ci], idx_v.at[q],
                              isems[q]).wait()

    def _start_gather(b, q):
        return pltpu.async_copy(h_hbm.at[idx_v.at[q, 0]], rows_v.at[b],
                                sems[b])

    def _drain_scatters(b, q):
        pltpu.make_async_copy(rows_v.at[b], out_sh.at[idx_v.at[q, 1]],
                              rsems[b]).wait()
        pltpu.make_async_copy(w_v.at[b], den_sh.at[idx_v.at[q, 1]],
                              wsems[b]).wait()

    def _process(ci, b, q, in_loop):
        def _d():
            _drain_scatters((ci - 2) % NBUF if not in_loop else (b + 1) % NBUF,
                            (ci - 2) % IQ if not in_loop else (q + 2) % IQ)
        if in_loop:
            pl.when(ci >= 2)(_d)
        elif ci >= 2:
            _d()
        def _g():
            _wait_load_idx(ci + 1, (q + 1) % IQ)
            _start_gather((b + 1) % NBUF, (q + 1) % IQ)
        def _p():
            _start_load_idx(ci + 2, (q + 2) % IQ)
        if in_loop:
            pl.when(ci + 1 < CHUNKS)(_g)
            pl.when(ci + 2 < CHUNKS)(_p)
        else:
            if ci + 1 < CHUNKS:
                _g()
            if ci + 2 < CHUNKS:
                _p()
        for g in range(K // 16):
            si = idx_v[q, 0, pl.ds(g * 16, 16)]
            di = idx_v[q, 1, pl.ds(g * 16, 16)]
            av = plsc.load_gather(tab_v, [si * 2])
            bv = plsc.load_gather(tab_v, [di * 2 + 1])
            e = av + bv
            e = jnp.maximum(e, 0.2 * e)
            w_v[b, pl.ds(g * 16, 16)] = jnp.exp(e)
        pltpu.make_async_copy(h_hbm.at[idx_v.at[q, 0]], rows_v.at[b],
                              sems[b]).wait()

        @plsc.parallel_loop(0, K, unroll=16)
        def _scale(j):
            wj = plsc.load_gather(w_v.at[b], [jnp.full((16,), j, jnp.int32)])
            for g in range(HID // 16):
                rows_v[b, j, pl.ds(g * 16, 16)] = (
                    rows_v[b, j, pl.ds(g * 16, 16)] * wj)

        pltpu.async_copy(rows_v.at[b], out_sh.at[idx_v.at[q, 1]], rsems[b],
                         add=True)
        pltpu.async_copy(w_v.at[b], den_sh.at[idx_v.at[q, 1]], wsems[b],
                         add=True)

    _start_load_idx(0, 0)
    _wait_load_idx(0, 0)
    _start_gather(0, 0)
    _start_load_idx(1, 1)

    STEP = 12
    nfull = (CHUNKS - 4) // STEP
    def _trip(t, _):
        base_ci = STEP * t
        for u in range(STEP):
            _process(base_ci + u, u % NBUF, u % IQ, True)
        return 0

    lax.fori_loop(0, nfull, _trip, 0)
    for ci in range(nfull * STEP, CHUNKS):
        _process(ci, ci % NBUF, ci % IQ, False)
    for ci in (CHUNKS - 2, CHUNKS - 1):
        _drain_scatters(ci % NBUF, ci % IQ)
    plsc.subcore_barrier()

    for off in range(0, STRIPE, K):
        sz = min(K, STRIPE - off)
        pltpu.sync_copy(out_sh.at[pl.ds(base + off, sz)],
                        outp_hbm.at[c, pl.ds(base + off, sz)])
    pltpu.sync_copy(den_sh.at[pl.ds(base, STRIPE)], denbuf_v)
    pltpu.sync_copy(denbuf_v, denp_hbm.at[pl.ds(c * NPAD + base, STRIPE)])


def _phase2(h, a_flat, ed):
    mesh = plsc.VectorSubcoreMesh(core_axis_name="c", subcore_axis_name="s")
    fn = pl.kernel(
        _sc_body,
        out_type=[
            jax.ShapeDtypeStruct((NC, NPAD, HID), jnp.float32),
            jax.ShapeDtypeStruct((NC * NPAD,), jnp.float32),
        ],
        mesh=mesh,
        compiler_params=pltpu.CompilerParams(needs_layout_passes=False),
        scratch_types=[
            pltpu.VMEM((2 * NPAD,), jnp.float32),
            pltpu.VMEM((IQ, 2, K), jnp.int32),
            pltpu.VMEM((NBUF, K), jnp.float32),
            pltpu.VMEM((NBUF, K, HID), jnp.float32),
            pltpu.VMEM((STRIPE,), jnp.float32),
            pltpu.VMEM_SHARED((NPAD, HID), jnp.float32),
            pltpu.VMEM_SHARED((NPAD,), jnp.float32),
        ] + [pltpu.SemaphoreType.DMA] * (3 * NBUF + IQ),
    )
    return fn(h, a_flat, ed)


def _p3_body(op_ref, dent_ref, a_ref, h_ref, b1_ref, w2_ref, b2_ref,
             o_ref):
    a_s = a_ref[:, 0]
    a_d = a_ref[:, 1]
    es = a_s + a_d
    es = jnp.maximum(es, 0.2 * es)
    w_self = jnp.exp(es)
    den = dent_ref[:, 0] + dent_ref[:, 1] + w_self + 1e-16
    out_un = op_ref[0] + op_ref[1] + w_self[:, None] * h_ref[...]
    h2 = out_un / den[:, None] + b1_ref[...]
    h2 = jnp.where(h2 > 0, h2, jnp.exp(h2) - 1.0)
    logits = jax.lax.dot_general(h2, w2_ref[...], (((1,), (0,)), ((), ())),
                                 preferred_element_type=jnp.float32)
    logits = logits + b2_ref[...]
    m = jnp.max(logits, axis=1, keepdims=True)
    z = logits - m
    o_ref[...] = z - jnp.log(jnp.sum(jnp.exp(z), axis=1, keepdims=True))


def _phase3(op, denT, a, h, b1, W2, b2):
    return pl.pallas_call(
        _p3_body,
        grid=(RB,),
        in_specs=[
            pl.BlockSpec((NC, R, HID), lambda i: (0, i, 0)),
            pl.BlockSpec((R, 2), lambda i: (i, 0)),
            pl.BlockSpec((R, 2), lambda i: (i, 0)),
            pl.BlockSpec((R, HID), lambda i: (i, 0)),
            pl.BlockSpec((1, HID), lambda i: (0, 0)),
            pl.BlockSpec((HID, OUT_C), lambda i: (0, 0)),
            pl.BlockSpec((1, OUT_C), lambda i: (0, 0)),
        ],
        out_specs=pl.BlockSpec((R, OUT_C), lambda i: (i, 0)),
        out_shape=jax.ShapeDtypeStruct((N, OUT_C), jnp.float32),
    )(op, denT, a, h, b1, W2, b2)


def kernel(x, edge_index, W1, att_src, att_dst, b1, W2, b2):
    att2 = jnp.stack([att_src, att_dst], axis=1)
    h, a = _phase1(x, W1, att2)

    pad = E_PAD - E
    pad_ar = jnp.arange(pad, dtype=jnp.int32)
    srcp = jnp.concatenate([edge_index[0], pad_ar % N])
    dstp = jnp.concatenate([edge_index[1], N + pad_ar % (NPAD - N)])
    ed = jnp.stack([srcp.reshape(NW * CHUNKS, K),
                    dstp.reshape(NW * CHUNKS, K)], axis=1)
    outp, denp = _phase2(h, a.reshape(2 * N), ed)

    denT = jnp.transpose(denp.reshape(NC, NPAD)[:, :N])
    return _phase3(outp, denT, a, h,
                   b1.reshape(1, HID), W2, b2.reshape(1, OUT_C))


# speedup vs baseline: 1.0739x; 1.0739x over previous
import functools

import jax
import jax.numpy as jnp
from jax import lax
from jax.experimental import pallas as pl
from jax.experimental.pallas import tpu as pltpu
from jax.experimental.pallas import tpu_sc as plsc

N = 10000
E = 320000
IN_C = 128
HID = 128
OUT_C = 64

NC = 2
NS = 16
NW = NC * NS
K = 64
CHUNKS = 160
NBUF = 3
IQ = 4
EPW = K * CHUNKS
E_PAD = EPW * NW
NPAD = 10112
STRIPE = NPAD // NS
RB = 25
R = N // RB


def _p1_body(x_ref, w1_ref, att2_ref, h_ref, a_ref):
    h = jax.lax.dot_general(x_ref[...], w1_ref[...], (((1,), (0,)), ((), ())),
                            preferred_element_type=jnp.float32)
    h_ref[...] = h
    a_ref[...] = jax.lax.dot_general(h, att2_ref[...], (((1,), (0,)), ((), ())),
                                     preferred_element_type=jnp.float32)


def _phase1(x, W1, att2):
    return pl.pallas_call(
        _p1_body,
        grid=(RB,),
        in_specs=[
            pl.BlockSpec((R, IN_C), lambda i: (i, 0)),
            pl.BlockSpec((IN_C, HID), lambda i: (0, 0)),
            pl.BlockSpec((HID, 2), lambda i: (0, 0)),
        ],
        out_specs=[
            pl.BlockSpec((R, HID), lambda i: (i, 0)),
            pl.BlockSpec((R, 2), lambda i: (i, 0)),
        ],
        out_shape=[
            jax.ShapeDtypeStruct((N, HID), jnp.float32),
            jax.ShapeDtypeStruct((N, 2), jnp.float32),
        ],
    )(x, W1, att2)


def _sc_body(h_hbm, a_hbm, ed_hbm,
             outp_hbm, denp_hbm,
             tab_v, idx_v, w_v, rows_v, denbuf_v,
             out_sh, den_sh,
             *sems9):
    c = lax.axis_index("c")
    s = lax.axis_index("s")
    wid = c * NS + s
    sems = sems9[0:NBUF]
    rsems = sems9[NBUF:2 * NBUF]
    wsems = sems9[2 * NBUF:3 * NBUF]
    isems = sems9[3 * NBUF:3 * NBUF + IQ]

    pltpu.sync_copy(a_hbm, tab_v.at[pl.ds(0, 2 * N)])
    z16 = jnp.zeros((16,), jnp.float32)
    for t in range(2 * N, 2 * NPAD, 16):
        tab_v[pl.ds(t, 16)] = z16

    def _zrow(j, _):
        for g in range(HID // 16):
            rows_v[0, j, pl.ds(g * 16, 16)] = z16
        return 0
    lax.fori_loop(0, K, _zrow, 0)
    base = s * STRIPE
    for off in range(0, STRIPE, K):
        sz = min(K, STRIPE - off)
        pltpu.sync_copy(rows_v.at[0, pl.ds(0, sz)],
                        out_sh.at[pl.ds(base + off, sz)])
        pltpu.sync_copy(rows_v.at[0, 0, pl.ds(0, sz)],
                        den_sh.at[pl.ds(base + off, sz)])
    plsc.subcore_barrier()

    def _start_load_idx(ci, q):
        pltpu.async_copy(ed_hbm.at[wid * CHUNKS + ci], idx_v.at[q], isems[q])

    def _wait_load_idx(ci, q):
        pltpu.make_async_copy(ed_hbm.at[wid * CHUNKS + ci], idx_v.at[q],
                              isems[q]).wait()

    def _start_gather(b, q):
        return pltpu.async_copy(h_hbm.at[idx_v.at[q, 0]], rows_v.at[b],
                                sems[b])

    def _drain_scatters(b, q):
        pltpu.make_async_copy(rows_v.at[b], out_sh.at[idx_v.at[q, 1]],
                              rsems[b]).wait()
        pltpu.make_async_copy(w_v.at[b], den_sh.at[idx_v.at[q, 1]],
                              wsems[b]).wait()

    def _process(ci, b, q, in_loop):
        def _d():
            _drain_scatters((ci - 2) % NBUF if not in_loop else (b + 1) % NBUF,
                            (ci - 2) % IQ if not in_loop else (q + 2) % IQ)
        if in_loop:
            pl.when(ci >= 2)(_d)
        elif ci >= 2:
            _d()
        def _g():
            _wait_load_idx(ci + 1, (q + 1) % IQ)
            _start_gather((b + 1) % NBUF, (q + 1) % IQ)
        def _p():
            _start_load_idx(ci + 2, (q + 2) % IQ)
        if in_loop:
            pl.when(ci + 1 < CHUNKS)(_g)
            pl.when(ci + 2 < CHUNKS)(_p)
        else:
            if ci + 1 < CHUNKS:
                _g()
            if ci + 2 < CHUNKS:
                _p()
        for g in range(K // 16):
            si = idx_v[q, 0, pl.ds(g * 16, 16)]
            di = idx_v[q, 1, pl.ds(g * 16, 16)]
            av = plsc.load_gather(tab_v, [si * 2])
            bv = plsc.load_gather(tab_v, [di * 2 + 1])
            e = av + bv
            e = jnp.maximum(e, 0.2 * e)
            w_v[b, pl.ds(g * 16, 16)] = jnp.exp(e)
        pltpu.make_async_copy(h_hbm.at[idx_v.at[q, 0]], rows_v.at[b],
                              sems[b]).wait()

        @plsc.parallel_loop(0, K, unroll=8)
        def _scale(j):
            wj = plsc.load_gather(w_v.at[b], [jnp.full((16,), j, jnp.int32)])
            for g in range(HID // 16):
                rows_v[b, j, pl.ds(g * 16, 16)] = (
                    rows_v[b, j, pl.ds(g * 16, 16)] * wj)

        pltpu.async_copy(rows_v.at[b], out_sh.at[idx_v.at[q, 1]], rsems[b],
                         add=True)
        pltpu.async_copy(w_v.at[b], den_sh.at[idx_v.at[q, 1]], wsems[b],
                         add=True)

    _start_load_idx(0, 0)
    _wait_load_idx(0, 0)
    _start_gather(0, 0)
    _start_load_idx(1, 1)

    STEP = 12
    nfull = (CHUNKS - 4) // STEP
    def _trip(t, _):
        base_ci = STEP * t
        for u in range(STEP):
            _process(base_ci + u, u % NBUF, u % IQ, True)
        return 0

    lax.fori_loop(0, nfull, _trip, 0)
    for ci in range(nfull * STEP, CHUNKS):
        _process(ci, ci % NBUF, ci % IQ, False)
    for ci in (CHUNKS - 2, CHUNKS - 1):
        _drain_scatters(ci % NBUF, ci % IQ)
    plsc.subcore_barrier()

    for off in range(0, STRIPE, K):
        sz = min(K, STRIPE - off)
        pltpu.sync_copy(out_sh.at[pl.ds(base + off, sz)],
                        outp_hbm.at[c, pl.ds(base + off, sz)])
    pltpu.sync_copy(den_sh.at[pl.ds(base, STRIPE)], denbuf_v)
    pltpu.sync_copy(denbuf_v, denp_hbm.at[pl.ds(c * NPAD + base, STRIPE)])


def _phase2(h, a_flat, ed):
    mesh = plsc.VectorSubcoreMesh(core_axis_name="c", subcore_axis_name="s")
    fn = pl.kernel(
        _sc_body,
        out_type=[
            jax.ShapeDtypeStruct((NC, NPAD, HID), jnp.float32),
            jax.ShapeDtypeStruct((NC * NPAD,), jnp.float32),
        ],
        mesh=mesh,
        compiler_params=pltpu.CompilerParams(needs_layout_passes=False),
        scratch_types=[
            pltpu.VMEM((2 * NPAD,), jnp.float32),
            pltpu.VMEM((IQ, 2, K), jnp.int32),
            pltpu.VMEM((NBUF, K), jnp.float32),
            pltpu.VMEM((NBUF, K, HID), jnp.float32),
            pltpu.VMEM((STRIPE,), jnp.float32),
            pltpu.VMEM_SHARED((NPAD, HID), jnp.float32),
            pltpu.VMEM_SHARED((NPAD,), jnp.float32),
        ] + [pltpu.SemaphoreType.DMA] * (3 * NBUF + IQ),
    )
    return fn(h, a_flat, ed)


def _p3_body(op_ref, dent_ref, a_ref, h_ref, b1_ref, w2_ref, b2_ref,
             o_ref):
    a_s = a_ref[:, 0]
    a_d = a_ref[:, 1]
    es = a_s + a_d
    es = jnp.maximum(es, 0.2 * es)
    w_self = jnp.exp(es)
    den = dent_ref[:, 0] + dent_ref[:, 1] + w_self + 1e-16
    out_un = op_ref[0] + op_ref[1] + w_self[:, None] * h_ref[...]
    h2 = out_un / den[:, None] + b1_ref[...]
    h2 = jnp.where(h2 > 0, h2, jnp.exp(h2) - 1.0)
    logits = jax.lax.dot_general(h2, w2_ref[...], (((1,), (0,)), ((), ())),
                                 preferred_element_type=jnp.float32)
    logits = logits + b2_ref[...]
    m = jnp.max(logits, axis=1, keepdims=True)
    z = logits - m
    o_ref[...] = z - jnp.log(jnp.sum(jnp.exp(z), axis=1, keepdims=True))


def _phase3(op, denT, a, h, b1, W2, b2):
    return pl.pallas_call(
        _p3_body,
        grid=(RB,),
        in_specs=[
            pl.BlockSpec((NC, R, HID), lambda i: (0, i, 0)),
            pl.BlockSpec((R, 2), lambda i: (i, 0)),
            pl.BlockSpec((R, 2), lambda i: (i, 0)),
            pl.BlockSpec((R, HID), lambda i: (i, 0)),
            pl.BlockSpec((1, HID), lambda i: (0, 0)),
            pl.BlockSpec((HID, OUT_C), lambda i: (0, 0)),
            pl.BlockSpec((1, OUT_C), lambda i: (0, 0)),
        ],
        out_specs=pl.BlockSpec((R, OUT_C), lambda i: (i, 0)),
        out_shape=jax.ShapeDtypeStruct((N, OUT_C), jnp.float32),
    )(op, denT, a, h, b1, W2, b2)


def kernel(x, edge_index, W1, att_src, att_dst, b1, W2, b2):
    att2 = jnp.stack([att_src, att_dst], axis=1)
    h, a = _phase1(x, W1, att2)

    pad = E_PAD - E
    pad_ar = jnp.arange(pad, dtype=jnp.int32)
    srcp = jnp.concatenate([edge_index[0], pad_ar % N])
    dstp = jnp.concatenate([edge_index[1], N + pad_ar % (NPAD - N)])
    ed = jnp.stack([srcp.reshape(NW * CHUNKS, K),
                    dstp.reshape(NW * CHUNKS, K)], axis=1)
    outp, denp = _phase2(h, a.reshape(2 * N), ed)

    denT = jnp.transpose(denp.reshape(NC, NPAD)[:, :N])
    return _phase3(outp, denT, a, h,
                   b1.reshape(1, HID), W2, b2.reshape(1, OUT_C))
